# baseline (speedup 1.0000x reference)
import jax
import jax.numpy as jnp
from jax import lax
from jax.experimental import pallas as pl
from jax.experimental.pallas import tpu as pltpu
from jax.experimental.pallas import tpu_sc as plsc

B, TQ, TK, D = 16, 2048, 2048, 512
G = 4
BG = B // G
NC, NS = 2, 16
NW = NC * NS
ROWS = B * TQ
ROWS_G = BG * TQ
ROWS_PER_W = ROWS_G // NW
CHUNK = 64
NCHUNK = ROWS_PER_W // CHUNK


def _make_argmax_body(batch0):
    def body(att_ref, idx_ref):
        b = pl.program_id(0)
        x = att_ref[0]
        m = jnp.max(x, axis=1, keepdims=True)
        it = lax.broadcasted_iota(jnp.int32, x.shape, 1)
        idx = jnp.min(jnp.where(x == m, it, TK), axis=1)
        idx_ref[0, 0, :] = idx + (batch0 + b) * TQ
    return body


def _flat_argmax(att_g, batch0):
    return pl.pallas_call(
        _make_argmax_body(batch0),
        grid=(BG,),
        in_specs=[pl.BlockSpec((1, TQ, TK), lambda b: (b, 0, 0))],
        out_specs=pl.BlockSpec((1, 1, TQ), lambda b: (b, 0, 0)),
        out_shape=jax.ShapeDtypeStruct((BG, 1, TQ), jnp.int32),
    )(att_g).reshape(ROWS_G)


def _gather_body(y_hbm, idx_hbm, out_hbm, idx_v, rows0, rows1,
                 gsem0, gsem1, ssem0, ssem1):
    wid = lax.axis_index("s") * NC + lax.axis_index("c")
    base = wid * ROWS_PER_W
    pltpu.sync_copy(idx_hbm.at[pl.ds(base, ROWS_PER_W)], idx_v)

    bufs = (rows0, rows1)
    gsems = (gsem0, gsem1)
    ssems = (ssem0, ssem1)

    def gather(c):
        return pltpu.make_async_copy(
            y_hbm.at[idx_v.at[pl.ds(c * CHUNK, CHUNK)]],
            bufs[c % 2], gsems[c % 2])

    def scatter(c):
        return pltpu.make_async_copy(
            bufs[c % 2], out_hbm.at[pl.ds(base + c * CHUNK, CHUNK)],
            ssems[c % 2])

    gather(0).start()
    for c in range(NCHUNK):
        if c + 1 < NCHUNK:
            if c >= 1:
                scatter(c - 1).wait()
            gather(c + 1).start()
        gather(c).wait()
        scatter(c).start()
    scatter(NCHUNK - 2).wait()
    scatter(NCHUNK - 1).wait()


def _sc_gather(y2d, flat_idx):
    mesh = plsc.VectorSubcoreMesh(core_axis_name="c", subcore_axis_name="s")
    f = pl.kernel(
        _gather_body,
        out_type=jax.ShapeDtypeStruct((ROWS_G, D), jnp.float32),
        mesh=mesh,
        scratch_types=[
            pltpu.VMEM((ROWS_PER_W,), jnp.int32),
            pltpu.VMEM((CHUNK, D), jnp.float32),
            pltpu.VMEM((CHUNK, D), jnp.float32),
            pltpu.SemaphoreType.DMA,
            pltpu.SemaphoreType.DMA,
            pltpu.SemaphoreType.DMA,
            pltpu.SemaphoreType.DMA,
        ],
    )
    return f(y2d, flat_idx)


@jax.jit
def kernel(y, att):
    y2d = y.reshape(ROWS, D)
    outs = []
    for g in range(G):
        flat_idx = _flat_argmax(att[g * BG:(g + 1) * BG], g * BG)
        outs.append(_sc_gather(y2d, flat_idx))
    return jnp.concatenate(outs, axis=0).reshape(B, TQ, D)


# speedup vs baseline: 1.7427x; 1.7427x over previous
import jax
import jax.numpy as jnp
from jax import lax
from jax.experimental import pallas as pl
from jax.experimental.pallas import tpu as pltpu
from jax.experimental.pallas import tpu_sc as plsc

B, TQ, TK, D = 16, 2048, 2048, 512
G = 4
BG = B // G
NC, NS = 2, 16
NW = NC * NS
ROWS = B * TQ
ROWS_G = BG * TQ
ROWS_PER_W = ROWS_G // NW
CHUNK = 64
NCHUNK = ROWS_PER_W // CHUNK


def _make_argmax_body(batch0):
    def body(att_ref, idx_ref):
        b = pl.program_id(0)
        x = att_ref[0]
        m = jnp.max(x, axis=1, keepdims=True)
        it = lax.broadcasted_iota(jnp.int32, x.shape, 1)
        idx = jnp.min(jnp.where(x == m, it, TK), axis=1)
        idx_ref[0, 0, :] = idx + (batch0 + b) * TQ
    return body


def _flat_argmax(att, batch0):
    return pl.pallas_call(
        _make_argmax_body(batch0),
        grid=(BG,),
        in_specs=[pl.BlockSpec((1, TQ, TK), lambda b, _b0=batch0: (_b0 + b, 0, 0))],
        out_specs=pl.BlockSpec((1, 1, TQ), lambda b: (b, 0, 0)),
        out_shape=jax.ShapeDtypeStruct((BG, 1, TQ), jnp.int32),
    )(att).reshape(ROWS_G)


def _gather_body(y_hbm, idx_hbm, out_hbm, idx_v, rows0, rows1,
                 gsem0, gsem1, ssem0, ssem1):
    wid = lax.axis_index("s") * NC + lax.axis_index("c")
    base = wid * ROWS_PER_W
    pltpu.sync_copy(idx_hbm.at[pl.ds(base, ROWS_PER_W)], idx_v)

    bufs = (rows0, rows1)
    gsems = (gsem0, gsem1)
    ssems = (ssem0, ssem1)

    def gather(c):
        return pltpu.make_async_copy(
            y_hbm.at[idx_v.at[pl.ds(c * CHUNK, CHUNK)]],
            bufs[c % 2], gsems[c % 2])

    def scatter(c):
        return pltpu.make_async_copy(
            bufs[c % 2], out_hbm.at[pl.ds(base + c * CHUNK, CHUNK)],
            ssems[c % 2])

    gather(0).start()
    for c in range(NCHUNK):
        if c + 1 < NCHUNK:
            if c >= 1:
                scatter(c - 1).wait()
            gather(c + 1).start()
        gather(c).wait()
        scatter(c).start()
    scatter(NCHUNK - 2).wait()
    scatter(NCHUNK - 1).wait()


def _sc_gather(y2d, flat_idx):
    mesh = plsc.VectorSubcoreMesh(core_axis_name="c", subcore_axis_name="s")
    f = pl.kernel(
        _gather_body,
        out_type=jax.ShapeDtypeStruct((ROWS_G, D), jnp.float32),
        mesh=mesh,
        scratch_types=[
            pltpu.VMEM((ROWS_PER_W,), jnp.int32),
            pltpu.VMEM((CHUNK, D), jnp.float32),
            pltpu.VMEM((CHUNK, D), jnp.float32),
            pltpu.SemaphoreType.DMA,
            pltpu.SemaphoreType.DMA,
            pltpu.SemaphoreType.DMA,
            pltpu.SemaphoreType.DMA,
        ],
    )
    return f(y2d, flat_idx)


@jax.jit
def kernel(y, att):
    y2d = y.reshape(ROWS, D)
    outs = []
    for g in range(G):
        flat_idx = _flat_argmax(att, g * BG)
        outs.append(_sc_gather(y2d, flat_idx))
    return jnp.concatenate(outs, axis=0).reshape(B, TQ, D)


# speedup vs baseline: 2.6011x; 1.4926x over previous
import jax
import jax.numpy as jnp
from jax import lax
from jax.experimental import pallas as pl
from jax.experimental.pallas import tpu as pltpu
from jax.experimental.pallas import tpu_sc as plsc

B, TQ, TK, D = 16, 2048, 2048, 512
NC, NS = 2, 16
NW = NC * NS
ROWS = B * TQ
ROWS_PER_W = ROWS // NW
CHUNK = 64
NCHUNK = ROWS_PER_W // CHUNK
NBUF = 3


def _argmax_body(att_ref, idx_ref):
    b = pl.program_id(0)
    x = att_ref[0]
    m = jnp.max(x, axis=1, keepdims=True)
    it = lax.broadcasted_iota(jnp.int32, x.shape, 1)
    idx = jnp.min(jnp.where(x == m, it, TK), axis=1)
    idx_ref[0, 0, :] = idx + b * TQ


def _flat_argmax(att):
    return pl.pallas_call(
        _argmax_body,
        grid=(B,),
        in_specs=[pl.BlockSpec((1, TQ, TK), lambda b: (b, 0, 0))],
        out_specs=pl.BlockSpec((1, 1, TQ), lambda b: (b, 0, 0)),
        out_shape=jax.ShapeDtypeStruct((B, 1, TQ), jnp.int32),
    )(att).reshape(ROWS)


def _gather_body(y_hbm, idx_hbm, out_hbm, idx_v, *bufs_and_sems):
    bufs = bufs_and_sems[:NBUF]
    gsems = bufs_and_sems[NBUF:2 * NBUF]
    ssems = bufs_and_sems[2 * NBUF:3 * NBUF]
    wid = lax.axis_index("s") * NC + lax.axis_index("c")
    base = wid * ROWS_PER_W
    pltpu.sync_copy(idx_hbm.at[pl.ds(base, ROWS_PER_W)], idx_v)

    def gather(c):
        return pltpu.make_async_copy(
            y_hbm.at[idx_v.at[pl.ds(c * CHUNK, CHUNK)]],
            bufs[c % NBUF], gsems[c % NBUF])

    def scatter(c):
        return pltpu.make_async_copy(
            bufs[c % NBUF], out_hbm.at[pl.ds(base + c * CHUNK, CHUNK)],
            ssems[c % NBUF])

    for k in range(NBUF - 1):
        gather(k).start()
    for c in range(NCHUNK):
        if c + NBUF - 1 < NCHUNK:
            if c >= 1:
                scatter(c - 1).wait()
            gather(c + NBUF - 1).start()
        gather(c).wait()
        scatter(c).start()
    for c in range(max(0, NCHUNK - NBUF), NCHUNK):
        scatter(c).wait()


def _sc_gather(y2d, flat_idx):
    mesh = plsc.VectorSubcoreMesh(core_axis_name="c", subcore_axis_name="s")
    f = pl.kernel(
        _gather_body,
        out_type=jax.ShapeDtypeStruct((ROWS, D), jnp.float32),
        mesh=mesh,
        scratch_types=(
            [pltpu.VMEM((ROWS_PER_W,), jnp.int32)]
            + [pltpu.VMEM((CHUNK, D), jnp.float32)] * NBUF
            + [pltpu.SemaphoreType.DMA] * (2 * NBUF)
        ),
    )
    return f(y2d, flat_idx)


@jax.jit
def kernel(y, att):
    flat_idx = _flat_argmax(att)
    out2d = _sc_gather(y.reshape(ROWS, D), flat_idx)
    return out2d.reshape(B, TQ, D)


# speedup vs baseline: 2.6114x; 1.0039x over previous
import jax
import jax.numpy as jnp
from jax import lax
from jax.experimental import pallas as pl
from jax.experimental.pallas import tpu as pltpu
from jax.experimental.pallas import tpu_sc as plsc

B, TQ, TK, D = 16, 2048, 2048, 512
NC, NS = 2, 16
NW = NC * NS
ROWS = B * TQ
ROWS_PER_W = ROWS // NW
CHUNK = 64
NCHUNK = ROWS_PER_W // CHUNK
NBUF = 3


def _argmax_body(att_ref, idx_ref):
    b = pl.program_id(0)
    x = att_ref[0]
    m = jnp.max(x, axis=1, keepdims=True)
    it = lax.broadcasted_iota(jnp.int32, x.shape, 1)
    idx = jnp.min(jnp.where(x == m, it, TK), axis=1)
    idx_ref[0, 0, :] = idx + b * TQ


def _flat_argmax(att):
    return pl.pallas_call(
        _argmax_body,
        grid=(B,),
        in_specs=[pl.BlockSpec((1, TQ, TK), lambda b: (b, 0, 0))],
        out_specs=pl.BlockSpec((1, 1, TQ), lambda b: (b, 0, 0)),
        out_shape=jax.ShapeDtypeStruct((B, 1, TQ), jnp.int32),
        compiler_params=pltpu.CompilerParams(skip_device_barrier=True),
    )(att).reshape(ROWS)


def _gather_body(y_hbm, idx_hbm, out_hbm, idx_v, *bufs_and_sems):
    bufs = bufs_and_sems[:NBUF]
    gsems = bufs_and_sems[NBUF:2 * NBUF]
    ssems = bufs_and_sems[2 * NBUF:3 * NBUF]
    wid = lax.axis_index("s") * NC + lax.axis_index("c")
    base = wid * ROWS_PER_W
    pltpu.sync_copy(idx_hbm.at[pl.ds(base, ROWS_PER_W)], idx_v)

    def gather(c):
        return pltpu.make_async_copy(
            y_hbm.at[idx_v.at[pl.ds(c * CHUNK, CHUNK)]],
            bufs[c % NBUF], gsems[c % NBUF])

    def scatter(c):
        return pltpu.make_async_copy(
            bufs[c % NBUF], out_hbm.at[pl.ds(base + c * CHUNK, CHUNK)],
            ssems[c % NBUF])

    for k in range(NBUF - 1):
        gather(k).start()
    for c in range(NCHUNK):
        if c + NBUF - 1 < NCHUNK:
            if c >= 1:
                scatter(c - 1).wait()
            gather(c + NBUF - 1).start()
        gather(c).wait()
        scatter(c).start()
    for c in range(max(0, NCHUNK - NBUF), NCHUNK):
        scatter(c).wait()


def _sc_gather(y2d, flat_idx):
    mesh = plsc.VectorSubcoreMesh(core_axis_name="c", subcore_axis_name="s")
    f = pl.kernel(
        _gather_body,
        out_type=jax.ShapeDtypeStruct((ROWS, D), jnp.float32),
        mesh=mesh,
        scratch_types=(
            [pltpu.VMEM((ROWS_PER_W,), jnp.int32)]
            + [pltpu.VMEM((CHUNK, D), jnp.float32)] * NBUF
            + [pltpu.SemaphoreType.DMA] * (2 * NBUF)
        ),
        compiler_params=pltpu.CompilerParams(skip_device_barrier=True),
    )
    return f(y2d, flat_idx)


@jax.jit
def kernel(y, att):
    flat_idx = _flat_argmax(att)
    out2d = _sc_gather(y.reshape(ROWS, D), flat_idx)
    return out2d.reshape(B, TQ, D)
